# baseline (speedup 1.0000x reference)
import functools

import jax
import jax.numpy as jnp
from jax import lax
from jax.experimental import pallas as pl
from jax.experimental.pallas import tpu as pltpu
from jax.experimental.pallas import tpu_sc as plsc

B = 8
N = 2048
SEGS = 2
SEG = N // SEGS
HALF = SEG // 2
V_PAD = 1024
L = 16
HCHUNKS = HALF // L


def _body(x_hbm, out_hbm, xv, ov, ha, hb, nb0, nb1, spm, sem):
    s = lax.axis_index("s")
    row = s // SEGS
    seg = s % SEGS
    base = row * N + seg * SEG

    in_cp = pltpu.async_copy(x_hbm.at[pl.ds(base, SEG)], xv, sem)

    def zero(i, _):
        ha[pl.ds(i * L, L)] = jnp.zeros((L,), jnp.float32)
        hb[pl.ds(i * L, L)] = jnp.zeros((L,), jnp.float32)
        return 0

    lax.fori_loop(0, V_PAD // L, zero, 0)
    in_cp.wait()

    def chunk(i, _):
        da = pl.ds(i * L, L)
        db = pl.ds(HALF + i * L, L)
        va = xv[da]
        vb = xv[db]
        preva = plsc.load_gather(ha, [va])
        prevb = plsc.load_gather(hb, [vb])
        ranka, lasta = plsc.scan_count(va)
        rankb, lastb = plsc.scan_count(vb)
        cnta = preva + ranka.astype(jnp.float32)
        cntb = prevb + rankb.astype(jnp.float32)
        ov[da] = cnta
        ov[db] = cntb
        plsc.store_scatter(ha, [va], cnta, mask=lasta)
        plsc.store_scatter(hb, [vb], cntb, mask=lastb)
        return 0

    lax.fori_loop(0, HCHUNKS, chunk, 0)

    @pl.when(seg == 0)
    def _():
        pa = pltpu.async_copy(ha, spm.at[2 * row], sem)
        pb = pltpu.async_copy(hb, spm.at[2 * row + 1], sem)

        def fix(i, _):
            d = pl.ds(HALF + i * L, L)
            ov[d] = ov[d] + plsc.load_gather(ha, [xv[d]])
            return 0

        lax.fori_loop(0, HCHUNKS, fix, 0)
        pa.wait()
        pb.wait()

    plsc.subcore_barrier()

    @pl.when(seg > 0)
    def _():
        ca = pltpu.async_copy(spm.at[2 * row], nb0, sem)
        cb = pltpu.async_copy(spm.at[2 * row + 1], nb1, sem)
        ca.wait()
        cb.wait()

        def off(i, _):
            d = pl.ds(i * L, L)
            v = xv[d]
            acc = ov[d] + plsc.load_gather(nb0, [v]) + plsc.load_gather(nb1, [v])
            own = plsc.load_gather(ha, [v])
            acc = acc + jnp.where(i >= HCHUNKS, own, jnp.zeros((L,), jnp.float32))
            ov[d] = acc
            return 0

        lax.fori_loop(0, 2 * HCHUNKS, off, 0)

    pltpu.sync_copy(ov, out_hbm.at[pl.ds(base, SEG)])


@jax.jit
def _counts(x):
    run = pl.kernel(
        _body,
        out_type=jax.ShapeDtypeStruct((B * N,), jnp.float32),
        mesh=plsc.VectorSubcoreMesh(
            core_axis_name="c", subcore_axis_name="s", num_cores=1
        ),
        scratch_types=[
            pltpu.VMEM((SEG,), jnp.int32),
            pltpu.VMEM((SEG,), jnp.float32),
            pltpu.VMEM((V_PAD,), jnp.float32),
            pltpu.VMEM((V_PAD,), jnp.float32),
            pltpu.VMEM((V_PAD,), jnp.float32),
            pltpu.VMEM((V_PAD,), jnp.float32),
            pltpu.VMEM_SHARED((16, V_PAD), jnp.float32),
            pltpu.SemaphoreType.DMA,
        ],
        compiler_params=pltpu.CompilerParams(needs_layout_passes=False),
    )
    return run(x.astype(jnp.int32).reshape(B * N))


def kernel(x):
    return _counts(x).reshape(B, N, 1)


# speedup vs baseline: 1.0215x; 1.0215x over previous
import functools

import jax
import jax.numpy as jnp
from jax import lax
from jax.experimental import pallas as pl
from jax.experimental.pallas import tpu as pltpu
from jax.experimental.pallas import tpu_sc as plsc

B = 8
N = 2048
SEGS = 2
SEG = N // SEGS
V_PAD = 1024
L = 16
CHUNKS = SEG // L


def _body(x_hbm, out_hbm, xv, ov, hist, nb0, spm, sem):
    s = lax.axis_index("s")
    row = s // SEGS
    seg = s % SEGS
    base = row * N + seg * SEG

    in_cp = pltpu.async_copy(x_hbm.at[pl.ds(base, SEG)], xv, sem)

    def zero(i, _):
        hist[pl.ds(i * L, L)] = jnp.zeros((L,), jnp.float32)
        return 0

    lax.fori_loop(0, V_PAD // L, zero, 0)
    in_cp.wait()

    def chunk(i, _):
        v = xv[pl.ds(i * L, L)]
        prev = plsc.load_gather(hist, [v])
        rank, last = plsc.scan_count(v)
        cnt = prev + rank.astype(jnp.float32)
        ov[pl.ds(i * L, L)] = cnt
        plsc.store_scatter(hist, [v], cnt, mask=last)
        return 0

    lax.fori_loop(0, CHUNKS, chunk, 0)

    @pl.when(seg == 0)
    def _():
        pltpu.sync_copy(hist, spm.at[s])

    plsc.subcore_barrier()

    @pl.when(seg > 0)
    def _():
        pltpu.sync_copy(spm.at[s - 1], nb0)

        def off(i, _):
            d = pl.ds(i * L, L)
            ov[d] = ov[d] + plsc.load_gather(nb0, [xv[d]])
            return 0

        lax.fori_loop(0, CHUNKS, off, 0)

    pltpu.sync_copy(ov, out_hbm.at[pl.ds(base, SEG)])


@jax.jit
def _counts(x):
    run = pl.kernel(
        _body,
        out_type=jax.ShapeDtypeStruct((B * N,), jnp.float32),
        mesh=plsc.VectorSubcoreMesh(
            core_axis_name="c", subcore_axis_name="s", num_cores=1
        ),
        scratch_types=[
            pltpu.VMEM((SEG,), jnp.int32),
            pltpu.VMEM((SEG,), jnp.float32),
            pltpu.VMEM((V_PAD,), jnp.float32),
            pltpu.VMEM((V_PAD,), jnp.float32),
            pltpu.VMEM_SHARED((16, V_PAD), jnp.float32),
            pltpu.SemaphoreType.DMA,
        ],
        compiler_params=pltpu.CompilerParams(
            needs_layout_passes=False, use_tc_tiling_on_sc=False
        ),
    )
    return run(x.astype(jnp.int32).reshape(B * N))


def kernel(x):
    return _counts(x).reshape(B, N, 1)
